# Initial kernel scaffold; baseline (speedup 1.0000x reference)
import jax
import jax.numpy as jnp
from jax.experimental import pallas as pl


def kernel(x, u, u_gt, edges):
    raise NotImplementedError("write your pallas kernel here")


# speedup vs baseline: 212.1141x; 212.1141x over previous
import functools

import jax
import jax.numpy as jnp
from jax import lax
from jax.experimental import pallas as pl
from jax.experimental.pallas import tpu as pltpu
from jax.experimental.pallas import tpu_sc as plsc

B, N, E = 2, 50000, 800000
NC, NS, L = 2, 16, 16
SB = 80
CJ = 16
NROW = E // SB
NCH = NROW // CJ
NG = -(-NCH // NS)
SG = N // 10
FN = 3200
FLAST = N - 15 * FN
DW = 8000
DWK = (B * N * 2) // DW
EPS = 1e-8
DIV_W, LAP_W, DATA_W = 1.0, 0.1, 1.0


def _rsqrt(z):
    ii = lax.bitcast_convert_type(z, jnp.int32)
    ii = jnp.int32(0x5F3759DF) - (ii >> 1)
    y = lax.bitcast_convert_type(ii, jnp.float32)
    for _ in range(3):
        y = y * (1.5 - 0.5 * z * y * y)
    return y


def _sc_body_real(x0f, x1f, u0f, u1f, uf, gf, zf, ei_hbm, ej_hbm, part_hbm,
                  t0, t1, t2, t3, a0, a1, a2, a3, a4,
                  idx_i, idx_j,
                  g0, g1, g2, g3, g4, g5, g6, g7,
                  b0, b1, b2, b3, b4, b5, b6, b7, b8, b9,
                  stg, f0, f1, f2, f3, f4, ub, gbv, pb,
                  sg, ss):
    c = lax.axis_index("c")
    s = lax.axis_index("s")
    wid = c * NS + s
    iot = lax.iota(jnp.int32, L)
    tabs = (t0, t1, t2, t3)
    accs = (a0, a1, a2, a3, a4)
    gb = (g0, g1, g2, g3, g4, g5, g6, g7)
    ubs = (b0, b1, b2, b3, b4, b5, b6, b7, b8, b9)
    srcs = (x0f, x1f, u0f, u1f)
    fins = (f0, f1, f2, f3, f4)

    @pl.when(s < 10)
    def _stage():
        for t in range(4):
            pltpu.sync_copy(srcs[t].at[pl.ds(c * N + s * SG, SG)], stg)
            pltpu.sync_copy(stg, tabs[t].at[pl.ds(s * SG, SG)])
        pltpu.sync_copy(zf, stg)
        for t in range(5):
            pltpu.sync_copy(stg, accs[t].at[pl.ds(s * SG, SG)])

    plsc.subcore_barrier()

    def _edge_sub(j, carry2):
        ir = idx_i.at[j]
        jr = idx_j.at[j]
        ds = []
        for t in range(4):
            ds.append(pltpu.async_copy(tabs[t].at[ir], gb[t], sg))
            ds.append(pltpu.async_copy(tabs[t].at[jr], gb[4 + t], sg))
        for d in ds:
            d.wait()
        for k in range(SB // L):
            sl = pl.ds(k * L, L)
            x0i = gb[0][sl]; x1i = gb[1][sl]
            u0i = gb[2][sl]; u1i = gb[3][sl]
            x0j = gb[4][sl]; x1j = gb[5][sl]
            u0j = gb[6][sl]; u1j = gb[7][sl]
            dx = x0j - x0i
            dy = x1j - x1i
            len2 = dx * dx + dy * dy + EPS
            r = _rsqrt(len2)
            wg = r * r
            rl = _rsqrt(len2 + EPS)
            wl = rl * rl
            du0 = u0j - u0i
            du1 = u1j - u1i
            divc = wg * r * (du0 * dx + du1 * dy)
            lap0 = wl * du0
            lap1 = wl * du1
            ubs[0][sl] = divc
            ubs[1][sl] = lap0
            ubs[2][sl] = lap1
            ubs[3][sl] = wg
            ubs[4][sl] = wl
            ubs[5][sl] = -divc
            ubs[6][sl] = -lap0
            ubs[7][sl] = -lap1
            ubs[8][sl] = wg
            ubs[9][sl] = wl
        ws = []
        for t in range(5):
            ws.append(pltpu.async_copy(ubs[t], accs[t].at[ir], ss, add=True))
            ws.append(pltpu.async_copy(ubs[5 + t], accs[t].at[jr], ss, add=True))
        for d in ws:
            d.wait()
        return carry2

    def edge_chunk(g, carry):
        h = g * NS + s

        @pl.when(h < NCH)
        def _chunk():
            pltpu.sync_copy(ei_hbm.at[c, pl.ds(h * CJ, CJ)], idx_i)
            pltpu.sync_copy(ej_hbm.at[c, pl.ds(h * CJ, CJ)], idx_j)
            lax.fori_loop(0, CJ, _edge_sub, 0)

        return carry

    lax.fori_loop(0, NG, edge_chunk, 0)
    plsc.subcore_barrier()

    @pl.when(s < 15)
    def _rb_full():
        for t in range(5):
            pltpu.sync_copy(accs[t].at[pl.ds(s * FN, FN)], fins[t])

    @pl.when(s == 15)
    def _rb_last():
        for t in range(5):
            pltpu.sync_copy(accs[t].at[pl.ds(15 * FN, FLAST)],
                            fins[t].at[pl.ds(0, FLAST)])

    limit = jnp.where(s < 15, FN, FLAST)
    zero = jnp.zeros((L,), jnp.float32)

    def nodef(t, carry):
        sdv, slp = carry
        rows0 = t * L + iot
        valid = (rows0 < limit).astype(jnp.float32)
        sl = pl.ds(t * L, L)
        a0v = f0[sl]; a1v = f1[sl]; a2v = f2[sl]
        a3v = f3[sl]; a4v = f4[sl]
        dv = a0v / (a3v + EPS)
        l0 = a1v / (a4v + EPS)
        l1 = a2v / (a4v + EPS)
        return (sdv + valid * dv * dv,
                slp + valid * (l0 * l0 + l1 * l1))

    sdv, slp = lax.fori_loop(0, FN // L, nodef, (zero, zero))

    base = jnp.minimum(wid, DWK - 1) * DW
    pltpu.sync_copy(uf.at[pl.ds(base, DW)], ub)
    pltpu.sync_copy(gf.at[pl.ds(base, DW)], gbv)

    def dataf(t, acc):
        dd = ub[pl.ds(t * L, L)] - gbv[pl.ds(t * L, L)]
        return acc + dd * dd

    sdat = lax.fori_loop(0, DW // L, dataf, zero)
    live = jnp.where(wid < DWK, 1.0, 0.0).astype(jnp.float32)

    pb[0] = sdv * (DIV_W / (B * N))
    pb[1] = slp * (LAP_W / (B * N * 2))
    pb[2] = sdat * live * (DATA_W / (B * N * 2))
    pltpu.sync_copy(pb, part_hbm.at[wid])


@functools.cache
def _build_sc_kernel():
    return pl.kernel(
        _sc_body_real,
        out_type=jax.ShapeDtypeStruct((NC * NS, 3, L), jnp.float32),
        mesh=plsc.VectorSubcoreMesh(
            core_axis_name="c", subcore_axis_name="s",
            num_cores=NC, num_subcores=NS),
        compiler_params=pltpu.CompilerParams(needs_layout_passes=False),
        scratch_types=(
            [pltpu.VMEM_SHARED((N,), jnp.float32)] * 4
            + [pltpu.VMEM_SHARED((N,), jnp.float32)] * 5
            + [pltpu.VMEM((CJ, SB), jnp.int32)] * 2
            + [pltpu.VMEM((SB,), jnp.float32)] * 8
            + [pltpu.VMEM((SB,), jnp.float32)] * 10
            + [pltpu.VMEM((SG,), jnp.float32)]
            + [pltpu.VMEM((FN,), jnp.float32)] * 5
            + [pltpu.VMEM((DW,), jnp.float32)] * 2
            + [pltpu.VMEM((3, L), jnp.float32)]
            + [pltpu.SemaphoreType.DMA] * 2
        ),
    )


def _sum_body(x_ref, o_ref):
    o_ref[0, 0] = jnp.sum(x_ref[...])


def _final_sum(x):
    return pl.pallas_call(
        _sum_body,
        out_shape=jax.ShapeDtypeStruct((1, 1), jnp.float32),
        out_specs=pl.BlockSpec(memory_space=pltpu.SMEM),
    )(x)


@jax.jit
def kernel(x, u, u_gt, edges):
    x0f = x[..., 0].reshape(-1)
    x1f = x[..., 1].reshape(-1)
    u0f = u[..., 0].reshape(-1)
    u1f = u[..., 1].reshape(-1)
    uf = u.reshape(-1)
    gf = u_gt.reshape(-1)
    zf = jnp.zeros((SG,), jnp.float32)
    ei = edges[..., 0].reshape(B, NROW, SB)
    ej = edges[..., 1].reshape(B, NROW, SB)
    part = _build_sc_kernel()(x0f, x1f, u0f, u1f, uf, gf, zf, ei, ej)
    return _final_sum(part.reshape(12, 128))[0, 0]


# speedup vs baseline: 263.0366x; 1.2401x over previous
import functools

import jax
import jax.numpy as jnp
from jax import lax
from jax.experimental import pallas as pl
from jax.experimental.pallas import tpu as pltpu
from jax.experimental.pallas import tpu_sc as plsc

B, N, E = 2, 50000, 800000
NC, NS, L = 2, 16, 16
SB = 80
CJ = 16
NROW = E // SB
NCH = NROW // CJ
NG = -(-NCH // NS)
SG = N // 10
FN = 3200
FLAST = N - 15 * FN
DW = 8000
DWK = (B * N * 2) // DW
EPS = 1e-8
DIV_W, LAP_W, DATA_W = 1.0, 0.1, 1.0


def _rsqrt(z):
    ii = lax.bitcast_convert_type(z, jnp.int32)
    ii = jnp.int32(0x5F3759DF) - (ii >> 1)
    y = lax.bitcast_convert_type(ii, jnp.float32)
    for _ in range(3):
        y = y * (1.5 - 0.5 * z * y * y)
    return y


def _sc_body_real(x0f, x1f, u0f, u1f, uf, gf, zf, ei_hbm, ej_hbm, part_hbm,
                  t0, t1, t2, t3, a0, a1, a2, a3, a4,
                  idx_i, idx_j,
                  g0, g1, g2, g3, g4, g5, g6, g7,
                  g8, g9, g10, g11, g12, g13, g14, g15,
                  b0, b1, b2, b3, b4, b5, b6, b7,
                  b8, b9, b10, b11, b12, b13, b14, b15,
                  stg, f0, f1, f2, f3, f4, ub, gbv, pb,
                  sg0, sg1, ss0, ss1):
    c = lax.axis_index("c")
    s = lax.axis_index("s")
    wid = c * NS + s
    iot = lax.iota(jnp.int32, L)
    tabs = (t0, t1, t2, t3)
    accs = (a0, a1, a2, a3, a4)
    gb = (g0, g1, g2, g3, g4, g5, g6, g7,
          g8, g9, g10, g11, g12, g13, g14, g15)
    ubs = (b0, b1, b2, b3, b4, b5, b6, b7,
           b8, b9, b10, b11, b12, b13, b14, b15)
    srcs = (x0f, x1f, u0f, u1f)
    fins = (f0, f1, f2, f3, f4)

    @pl.when(s < 10)
    def _stage():
        for t in range(4):
            pltpu.sync_copy(srcs[t].at[pl.ds(c * N + s * SG, SG)], stg)
            pltpu.sync_copy(stg, tabs[t].at[pl.ds(s * SG, SG)])
        pltpu.sync_copy(zf, stg)
        for t in range(5):
            pltpu.sync_copy(stg, accs[t].at[pl.ds(s * SG, SG)])

    plsc.subcore_barrier()

    sgs = (sg0, sg1)
    sss = (ss0, ss1)
    dummy = x0f.at[pl.ds(0, SB)]

    def issue_gathers(p, j):
        ir = idx_i.at[j]
        jr = idx_j.at[j]
        for t in range(4):
            pltpu.async_copy(tabs[t].at[ir], gb[8 * p + t], sgs[p])
            pltpu.async_copy(tabs[t].at[jr], gb[8 * p + 4 + t], sgs[p])

    def wait_gathers(p):
        for t in range(8):
            pltpu.make_async_copy(dummy, gb[8 * p + t], sgs[p]).wait()

    def issue_scatters(p, j):
        ir = idx_i.at[j]
        jr = idx_j.at[j]
        o = 8 * p
        for t in range(3):
            pltpu.async_copy(ubs[o + t], accs[t].at[ir], sss[p], add=True)
            pltpu.async_copy(ubs[o + 5 + t], accs[t].at[jr], sss[p], add=True)
        for t in range(3, 5):
            pltpu.async_copy(ubs[o + t], accs[t].at[ir], sss[p], add=True)
            pltpu.async_copy(ubs[o + t], accs[t].at[jr], sss[p], add=True)

    def wait_scatters(p):
        for t in range(10):
            pltpu.make_async_copy(dummy, ubs[8 * p], sss[p]).wait()

    def compute(p):
        o = 8 * p
        for k in range(SB // L):
            sl = pl.ds(k * L, L)
            x0i = gb[o + 0][sl]; x1i = gb[o + 1][sl]
            u0i = gb[o + 2][sl]; u1i = gb[o + 3][sl]
            x0j = gb[o + 4][sl]; x1j = gb[o + 5][sl]
            u0j = gb[o + 6][sl]; u1j = gb[o + 7][sl]
            dx = x0j - x0i
            dy = x1j - x1i
            len2 = dx * dx + dy * dy + EPS
            r = _rsqrt(len2)
            wg = r * r
            rl = _rsqrt(len2 + EPS)
            wl = rl * rl
            du0 = u0j - u0i
            du1 = u1j - u1i
            divc = wg * r * (du0 * dx + du1 * dy)
            lap0 = wl * du0
            lap1 = wl * du1
            ubs[o + 0][sl] = divc
            ubs[o + 1][sl] = lap0
            ubs[o + 2][sl] = lap1
            ubs[o + 3][sl] = wg
            ubs[o + 4][sl] = wl
            ubs[o + 5][sl] = -divc
            ubs[o + 6][sl] = -lap0
            ubs[o + 7][sl] = -lap1

    def _pipe_pair(jj, carry2):
        j = 2 * jj
        wait_gathers(0)
        issue_gathers(1, j + 1)

        @pl.when(jj > 0)
        def _():
            wait_scatters(0)

        compute(0)
        issue_scatters(0, j)

        wait_gathers(1)

        @pl.when(jj < CJ // 2 - 1)
        def _():
            issue_gathers(0, j + 2)

        @pl.when(jj > 0)
        def _():
            wait_scatters(1)

        compute(1)
        issue_scatters(1, j + 1)
        return carry2

    def edge_chunk(g, carry):
        h = g * NS + s

        @pl.when(h < NCH)
        def _chunk():
            pltpu.sync_copy(ei_hbm.at[c, pl.ds(h * CJ, CJ)], idx_i)
            pltpu.sync_copy(ej_hbm.at[c, pl.ds(h * CJ, CJ)], idx_j)
            issue_gathers(0, 0)
            lax.fori_loop(0, CJ // 2, _pipe_pair, 0)
            wait_scatters(0)
            wait_scatters(1)

        return carry

    lax.fori_loop(0, NG, edge_chunk, 0)
    plsc.subcore_barrier()

    @pl.when(s < 15)
    def _rb_full():
        for t in range(5):
            pltpu.sync_copy(accs[t].at[pl.ds(s * FN, FN)], fins[t])

    @pl.when(s == 15)
    def _rb_last():
        for t in range(5):
            pltpu.sync_copy(accs[t].at[pl.ds(15 * FN, FLAST)],
                            fins[t].at[pl.ds(0, FLAST)])

    limit = jnp.where(s < 15, FN, FLAST)
    zero = jnp.zeros((L,), jnp.float32)

    def nodef(t, carry):
        sdv, slp = carry
        rows0 = t * L + iot
        valid = (rows0 < limit).astype(jnp.float32)
        sl = pl.ds(t * L, L)
        a0v = f0[sl]; a1v = f1[sl]; a2v = f2[sl]
        a3v = f3[sl]; a4v = f4[sl]
        dv = a0v / (a3v + EPS)
        l0 = a1v / (a4v + EPS)
        l1 = a2v / (a4v + EPS)
        return (sdv + valid * dv * dv,
                slp + valid * (l0 * l0 + l1 * l1))

    sdv, slp = lax.fori_loop(0, FN // L, nodef, (zero, zero))

    base = jnp.minimum(wid, DWK - 1) * DW
    pltpu.sync_copy(uf.at[pl.ds(base, DW)], ub)
    pltpu.sync_copy(gf.at[pl.ds(base, DW)], gbv)

    def dataf(t, acc):
        dd = ub[pl.ds(t * L, L)] - gbv[pl.ds(t * L, L)]
        return acc + dd * dd

    sdat = lax.fori_loop(0, DW // L, dataf, zero)
    live = jnp.where(wid < DWK, 1.0, 0.0).astype(jnp.float32)

    pb[0] = sdv * (DIV_W / (B * N))
    pb[1] = slp * (LAP_W / (B * N * 2))
    pb[2] = sdat * live * (DATA_W / (B * N * 2))
    pltpu.sync_copy(pb, part_hbm.at[wid])


@functools.cache
def _build_sc_kernel():
    return pl.kernel(
        _sc_body_real,
        out_type=jax.ShapeDtypeStruct((NC * NS, 3, L), jnp.float32),
        mesh=plsc.VectorSubcoreMesh(
            core_axis_name="c", subcore_axis_name="s",
            num_cores=NC, num_subcores=NS),
        compiler_params=pltpu.CompilerParams(needs_layout_passes=False),
        scratch_types=(
            [pltpu.VMEM_SHARED((N,), jnp.float32)] * 4
            + [pltpu.VMEM_SHARED((N,), jnp.float32)] * 5
            + [pltpu.VMEM((CJ, SB), jnp.int32)] * 2
            + [pltpu.VMEM((SB,), jnp.float32)] * 16
            + [pltpu.VMEM((SB,), jnp.float32)] * 16
            + [pltpu.VMEM((SG,), jnp.float32)]
            + [pltpu.VMEM((FN,), jnp.float32)] * 5
            + [pltpu.VMEM((DW,), jnp.float32)] * 2
            + [pltpu.VMEM((3, L), jnp.float32)]
            + [pltpu.SemaphoreType.DMA] * 4
        ),
    )


def _sum_body(x_ref, o_ref):
    o_ref[0, 0] = jnp.sum(x_ref[...])


def _final_sum(x):
    return pl.pallas_call(
        _sum_body,
        out_shape=jax.ShapeDtypeStruct((1, 1), jnp.float32),
        out_specs=pl.BlockSpec(memory_space=pltpu.SMEM),
    )(x)


@jax.jit
def kernel(x, u, u_gt, edges):
    x0f = x[..., 0].reshape(-1)
    x1f = x[..., 1].reshape(-1)
    u0f = u[..., 0].reshape(-1)
    u1f = u[..., 1].reshape(-1)
    uf = u.reshape(-1)
    gf = u_gt.reshape(-1)
    zf = jnp.zeros((SG,), jnp.float32)
    ei = edges[..., 0].reshape(B, NROW, SB)
    ej = edges[..., 1].reshape(B, NROW, SB)
    part = _build_sc_kernel()(x0f, x1f, u0f, u1f, uf, gf, zf, ei, ej)
    return _final_sum(part.reshape(12, 128))[0, 0]
